# baseline (speedup 1.0000x reference)
import math

import numpy as np
import jax
import jax.numpy as jnp
from jax import lax
from jax.experimental import pallas as pl
from jax.experimental.pallas import tpu as pltpu

_B = 2
_H = 224
_W = 224
_H0 = 64
_C = 160
_ROWS_PER_TILE = 32
_TILE_TOKENS = _ROWS_PER_TILE * _W
_TILES_PER_BATCH = _H // _ROWS_PER_TILE
_NUM_TILES = _B * _TILES_PER_BATCH
_K = int(_B * _H * _W * 0.2)
_FINE_TILES = -(-_K // _TILE_TOKENS)
_NUM_FREQ = 10
_MAX_FREQ = 10.0
_POS_DIM = 2 * _NUM_FREQ * 2 + 2
_EPS = 1e-5


def _tile_kernel(emb_ref, pe_ref, coords_ref, m_ref,
                 w0a_ref, b0a_ref, w0b_ref, b0b_ref,
                 wh0_ref, bh0_ref, whr_ref, bhr_ref,
                 w1a_ref, b1a_ref, w1b_ref, b1b_ref,
                 w3a_ref, b3a_ref, w3b_ref, b3b_ref,
                 coarse_ref, fine_ref, v_scr):
    i = pl.program_id(0)
    rb = lax.rem(i, _TILES_PER_BATCH)
    r0 = rb * _ROWS_PER_TILE
    m_full = m_ref[...]

    @pl.when(rb == 0)
    def _vert():
        img = jnp.concatenate([emb_ref[0], pe_ref[0]],
                              axis=0).astype(jnp.bfloat16)
        v_scr[...] = lax.dot_general(
            m_full, img, (((1,), (1,)), ((), ())),
            preferred_element_type=jnp.float32).astype(jnp.bfloat16)

    t_tile = v_scr[pl.ds(r0, _ROWS_PER_TILE)]
    if2 = lax.dot_general(m_full, t_tile, (((1,), (2,)), ((), ())),
                          preferred_element_type=jnp.float32)
    img_f = if2.reshape(_TILE_TOKENS, _C)

    feat = jnp.concatenate([img_f, coords_ref[0]], axis=1)
    feat = feat.astype(jnp.bfloat16)

    def ldot(x, w_ref, b_ref):
        r = lax.dot_general(x, w_ref[...], (((1,), (0,)), ((), ())),
                            preferred_element_type=jnp.float32) + b_ref[...]
        return r

    def ldot16(x, w_ref, b_ref):
        return ldot(x, w_ref, b_ref).astype(jnp.bfloat16)

    h = jax.nn.relu(ldot16(feat, w0a_ref, b0a_ref))
    h = jax.nn.relu(ldot16(h, w0b_ref, b0b_ref))
    s0 = ldot(h, wh0_ref, bh0_ref)

    s0_t = jnp.transpose(s0.reshape(_W, _ROWS_PER_TILE), (1, 0))
    coarse_ref[...] = s0_t[None]

    @pl.when(i < _FINE_TILES)
    def _fine():
        s_rest = ldot16(h, whr_ref, bhr_ref)
        fine_in = jnp.concatenate([feat, s_rest], axis=1)
        y = jax.nn.relu(ldot16(fine_in, w1a_ref, b1a_ref))
        y = jax.nn.relu(ldot16(y, w1b_ref, b1b_ref))
        z = jax.nn.relu(ldot16(y, w3a_ref, b3a_ref))
        z = ldot(z, w3b_ref, b3b_ref)
        z_t = jnp.transpose(z.reshape(_W, _ROWS_PER_TILE), (1, 0))
        tid = i * _TILE_TOKENS \
            + lax.broadcasted_iota(jnp.int32, (_ROWS_PER_TILE, _W), 0) * _W \
            + lax.broadcasted_iota(jnp.int32, (_ROWS_PER_TILE, _W), 1)
        fine_ref[...] = jnp.where(tid < _K, z_t, s0_t)[None]

    @pl.when(i >= _FINE_TILES)
    def _copy():
        fine_ref[...] = s0_t[None]


def _fold(lin, bn):
    scale = bn['g'] / jnp.sqrt(bn['v'] + _EPS)
    w = lin['W'] * scale[None, :]
    b = (lin['b'] - bn['m']) * scale + bn['be']
    return w.astype(jnp.bfloat16), b.astype(jnp.float32).reshape(1, -1)


def _coords_table():
    gy = jnp.linspace(-1.0, 1.0, _H)
    gx = jnp.linspace(-1.0, 1.0, _W)
    yy = jnp.broadcast_to(gy[:, None], (_H, _W))
    xx = jnp.broadcast_to(gx[None, :], (_H, _W))
    coords = jnp.stack([yy, xx], axis=-1)
    freqs = 2.0 ** jnp.linspace(0.0, _MAX_FREQ, _NUM_FREQ)
    si = (2.0 * math.pi * coords[..., None] * freqs).reshape(_H, _W, -1)
    enc = jnp.concatenate([jnp.sin(si), jnp.sin(si + math.pi / 2.0), coords],
                          axis=-1).astype(jnp.float32)
    t = enc.reshape(_TILES_PER_BATCH, _ROWS_PER_TILE, _W, _POS_DIM)
    return jnp.transpose(t, (0, 2, 1, 3)).reshape(
        _TILES_PER_BATCH, _TILE_TOKENS, _POS_DIM)


@jax.jit
def _run(image_embedding, image_pe, params):
    p = params
    w0a, b0a = _fold(p['l0a'], p['bn0a'])
    w0b, b0b = _fold(p['l0b'], p['bn0b'])
    wh = p['head']['W']
    bh = p['head']['b'].astype(jnp.float32)
    wh0, bh0 = wh[:, 0:1].astype(jnp.bfloat16), bh[0:1].reshape(1, 1)
    whr, bhr = wh[:, 1:].astype(jnp.bfloat16), bh[1:].reshape(1, -1)
    w1a, b1a = _fold(p['l1a'], p['bn1a'])
    w1b, b1b = _fold(p['l1b'], p['bn1b'])
    w3a, b3a = _fold(p['l3a'], p['bn3a'])
    w3b = p['l3b']['W'].astype(jnp.bfloat16)
    b3b = p['l3b']['b'].astype(jnp.float32).reshape(1, 1)

    m = jax.image.resize(jnp.eye(_H0, dtype=jnp.float32), (_H, _H0),
                         method='bilinear').astype(jnp.bfloat16)
    coords = _coords_table()
    emb = image_embedding.astype(jnp.float32)
    pe = image_pe.astype(jnp.float32)

    def whole(a):
        return pl.BlockSpec(a.shape, lambda i: (0,) * a.ndim)

    grid = (_NUM_TILES,)
    in_specs = [
        pl.BlockSpec((1, 32, _H0, _H0), lambda i: (i // _TILES_PER_BATCH, 0, 0, 0)),
        pl.BlockSpec((1, 128, _H0, _H0), lambda i: (i // _TILES_PER_BATCH, 0, 0, 0)),
        pl.BlockSpec((1, _TILE_TOKENS, _POS_DIM),
                     lambda i: (lax.rem(i, _TILES_PER_BATCH), 0, 0)),
        whole(m),
        whole(w0a), whole(b0a), whole(w0b), whole(b0b),
        whole(wh0), whole(bh0), whole(whr), whole(bhr),
        whole(w1a), whole(b1a), whole(w1b), whole(b1b),
        whole(w3a), whole(b3a), whole(w3b), whole(b3b),
    ]
    out_spec = pl.BlockSpec((1, _ROWS_PER_TILE, _W),
                            lambda i: (i // _TILES_PER_BATCH,
                                       lax.rem(i, _TILES_PER_BATCH), 0))
    coarse, fine = pl.pallas_call(
        _tile_kernel,
        grid=grid,
        in_specs=in_specs,
        out_specs=[out_spec, out_spec],
        out_shape=[jax.ShapeDtypeStruct((_B, _H, _W), jnp.float32)] * 2,
        scratch_shapes=[pltpu.VMEM((_H, _C, _H0), jnp.bfloat16)],
    )(emb, pe, coords, m, w0a, b0a, w0b, b0b, wh0, bh0, whr, bhr,
      w1a, b1a, w1b, b1b, w3a, b3a, w3b, b3b)
    return (coarse.reshape(_B, 1, _H, _W), fine.reshape(_B, 1, _H, _W))


def kernel(image_embedding, image_pe, params, original_shape):
    del original_shape
    return _run(image_embedding, image_pe, params)


# speedup vs baseline: 1.0556x; 1.0556x over previous
import math

import numpy as np
import jax
import jax.numpy as jnp
from jax import lax
from jax.experimental import pallas as pl
from jax.experimental.pallas import tpu as pltpu

_B = 2
_H = 224
_W = 224
_H0 = 64
_C = 160
_ROWS_PER_TILE = 16
_TILE_TOKENS = _ROWS_PER_TILE * _W
_TILES_PER_BATCH = _H // _ROWS_PER_TILE
_NUM_TILES = _B * _TILES_PER_BATCH
_K = int(_B * _H * _W * 0.2)
_FINE_TILES = -(-_K // _TILE_TOKENS)
_NUM_FREQ = 10
_MAX_FREQ = 10.0
_POS_DIM = 2 * _NUM_FREQ * 2 + 2
_EPS = 1e-5


def _tile_kernel(emb_ref, pe_ref, coords_ref, m_ref,
                 w0a_ref, b0a_ref, w0b_ref, b0b_ref,
                 wh0_ref, bh0_ref, whr_ref, bhr_ref,
                 w1a_ref, b1a_ref, w1b_ref, b1b_ref,
                 w3a_ref, b3a_ref, w3b_ref, b3b_ref,
                 coarse_ref, fine_ref, v_scr):
    i = pl.program_id(0)
    rb = lax.rem(i, _TILES_PER_BATCH)
    r0 = rb * _ROWS_PER_TILE
    m_full = m_ref[...]

    @pl.when(rb == 0)
    def _vert():
        img = jnp.concatenate([emb_ref[0], pe_ref[0]],
                              axis=0).astype(jnp.bfloat16)
        v_scr[...] = lax.dot_general(
            m_full, img, (((1,), (1,)), ((), ())),
            preferred_element_type=jnp.float32).astype(jnp.bfloat16)

    t_tile = v_scr[pl.ds(r0, _ROWS_PER_TILE)]
    if2 = lax.dot_general(m_full, t_tile, (((1,), (2,)), ((), ())),
                          preferred_element_type=jnp.float32
                          ).astype(jnp.bfloat16)
    img_f = if2.reshape(_TILE_TOKENS, _C)

    feat = jnp.concatenate([img_f, coords_ref[0]], axis=1)

    def ldot(x, w_ref, b_ref):
        r = lax.dot_general(x, w_ref[...], (((1,), (0,)), ((), ())),
                            preferred_element_type=jnp.float32) + b_ref[...]
        return r

    def ldot16(x, w_ref, b_ref):
        return ldot(x, w_ref, b_ref).astype(jnp.bfloat16)

    h = jax.nn.relu(ldot16(feat, w0a_ref, b0a_ref))
    h = jax.nn.relu(ldot16(h, w0b_ref, b0b_ref))
    s0 = ldot(h, wh0_ref, bh0_ref)

    s0_t = jnp.transpose(s0.reshape(_W, _ROWS_PER_TILE), (1, 0))
    coarse_ref[...] = s0_t[None]

    @pl.when(i < _FINE_TILES)
    def _fine():
        s_rest = ldot16(h, whr_ref, bhr_ref)
        fine_in = jnp.concatenate([feat, s_rest], axis=1)
        y = jax.nn.relu(ldot16(fine_in, w1a_ref, b1a_ref))
        y = jax.nn.relu(ldot16(y, w1b_ref, b1b_ref))
        z = jax.nn.relu(ldot16(y, w3a_ref, b3a_ref))
        z = ldot(z, w3b_ref, b3b_ref)
        z_t = jnp.transpose(z.reshape(_W, _ROWS_PER_TILE), (1, 0))
        tid = i * _TILE_TOKENS \
            + lax.broadcasted_iota(jnp.int32, (_ROWS_PER_TILE, _W), 0) * _W \
            + lax.broadcasted_iota(jnp.int32, (_ROWS_PER_TILE, _W), 1)
        fine_ref[...] = jnp.where(tid < _K, z_t, s0_t)[None]

    @pl.when(i >= _FINE_TILES)
    def _copy():
        fine_ref[...] = s0_t[None]


def _fold(lin, bn):
    scale = bn['g'] / jnp.sqrt(bn['v'] + _EPS)
    w = lin['W'] * scale[None, :]
    b = (lin['b'] - bn['m']) * scale + bn['be']
    return w.astype(jnp.bfloat16), b.astype(jnp.float32).reshape(1, -1)


def _coords_table():
    gy = jnp.linspace(-1.0, 1.0, _H)
    gx = jnp.linspace(-1.0, 1.0, _W)
    yy = jnp.broadcast_to(gy[:, None], (_H, _W))
    xx = jnp.broadcast_to(gx[None, :], (_H, _W))
    coords = jnp.stack([yy, xx], axis=-1)
    freqs = 2.0 ** jnp.linspace(0.0, _MAX_FREQ, _NUM_FREQ)
    si = (2.0 * math.pi * coords[..., None] * freqs).reshape(_H, _W, -1)
    enc = jnp.concatenate([jnp.sin(si), jnp.sin(si + math.pi / 2.0), coords],
                          axis=-1).astype(jnp.bfloat16)
    t = enc.reshape(_TILES_PER_BATCH, _ROWS_PER_TILE, _W, _POS_DIM)
    return jnp.transpose(t, (0, 2, 1, 3)).reshape(
        _TILES_PER_BATCH, _TILE_TOKENS, _POS_DIM)


@jax.jit
def _run(image_embedding, image_pe, params):
    p = params
    w0a, b0a = _fold(p['l0a'], p['bn0a'])
    w0b, b0b = _fold(p['l0b'], p['bn0b'])
    wh = p['head']['W']
    bh = p['head']['b'].astype(jnp.float32)
    wh0, bh0 = wh[:, 0:1].astype(jnp.bfloat16), bh[0:1].reshape(1, 1)
    whr, bhr = wh[:, 1:].astype(jnp.bfloat16), bh[1:].reshape(1, -1)
    w1a, b1a = _fold(p['l1a'], p['bn1a'])
    w1b, b1b = _fold(p['l1b'], p['bn1b'])
    w3a, b3a = _fold(p['l3a'], p['bn3a'])
    w3b = p['l3b']['W'].astype(jnp.bfloat16)
    b3b = p['l3b']['b'].astype(jnp.float32).reshape(1, 1)

    m = jax.image.resize(jnp.eye(_H0, dtype=jnp.float32), (_H, _H0),
                         method='bilinear').astype(jnp.bfloat16)
    coords = _coords_table()
    emb = image_embedding.astype(jnp.float32)
    pe = image_pe.astype(jnp.float32)

    def whole(a):
        return pl.BlockSpec(a.shape, lambda i: (0,) * a.ndim)

    grid = (_NUM_TILES,)
    in_specs = [
        pl.BlockSpec((1, 32, _H0, _H0), lambda i: (i // _TILES_PER_BATCH, 0, 0, 0)),
        pl.BlockSpec((1, 128, _H0, _H0), lambda i: (i // _TILES_PER_BATCH, 0, 0, 0)),
        pl.BlockSpec((1, _TILE_TOKENS, _POS_DIM),
                     lambda i: (lax.rem(i, _TILES_PER_BATCH), 0, 0)),
        whole(m),
        whole(w0a), whole(b0a), whole(w0b), whole(b0b),
        whole(wh0), whole(bh0), whole(whr), whole(bhr),
        whole(w1a), whole(b1a), whole(w1b), whole(b1b),
        whole(w3a), whole(b3a), whole(w3b), whole(b3b),
    ]
    out_spec = pl.BlockSpec((1, _ROWS_PER_TILE, _W),
                            lambda i: (i // _TILES_PER_BATCH,
                                       lax.rem(i, _TILES_PER_BATCH), 0))
    coarse, fine = pl.pallas_call(
        _tile_kernel,
        grid=grid,
        in_specs=in_specs,
        out_specs=[out_spec, out_spec],
        out_shape=[jax.ShapeDtypeStruct((_B, _H, _W), jnp.float32)] * 2,
        scratch_shapes=[pltpu.VMEM((_H, _C, _H0), jnp.bfloat16)],
    )(emb, pe, coords, m, w0a, b0a, w0b, b0b, wh0, bh0, whr, bhr,
      w1a, b1a, w1b, b1b, w3a, b3a, w3b, b3b)
    return (coarse.reshape(_B, 1, _H, _W), fine.reshape(_B, 1, _H, _W))


def kernel(image_embedding, image_pe, params, original_shape):
    del original_shape
    return _run(image_embedding, image_pe, params)


# speedup vs baseline: 1.0560x; 1.0004x over previous
import math

import jax
import jax.numpy as jnp
from jax import lax
from jax.experimental import pallas as pl
from jax.experimental.pallas import tpu as pltpu

_B = 2
_H = 224
_W = 224
_H0 = 64
_C = 160
_ROWS_PER_TILE = 16
_TILE_TOKENS = _ROWS_PER_TILE * _W
_TILES_PER_BATCH = _H // _ROWS_PER_TILE
_NUM_TILES = _B * _TILES_PER_BATCH
_K = int(_B * _H * _W * 0.2)
_FINE_TILES = -(-_K // _TILE_TOKENS)
_NUM_FREQ = 10
_MAX_FREQ = 10.0
_POS_DIM = 2 * _NUM_FREQ * 2 + 2
_EPS = 1e-5


def _tile_kernel(emb_ref, pe_ref, coords_ref, m_ref,
                 w0a_ref, b0a_ref, w0b_ref, b0b_ref,
                 wh0_ref, bh0_ref, whr_ref, bhr_ref,
                 w1a_ref, b1a_ref, w1b_ref, b1b_ref,
                 w3a_ref, b3a_ref, w3b_ref, b3b_ref,
                 coarse_ref, fine_ref, v_scr):
    i = pl.program_id(0)
    rb = lax.rem(i, _TILES_PER_BATCH)
    r0 = rb * _ROWS_PER_TILE
    m_full = m_ref[...]

    @pl.when(rb == 0)
    def _vert():
        img = jnp.concatenate([emb_ref[0], pe_ref[0]],
                              axis=0).astype(jnp.bfloat16)
        v_scr[...] = lax.dot_general(
            m_full, img, (((1,), (1,)), ((), ())),
            preferred_element_type=jnp.float32).astype(jnp.bfloat16)

    t_tile = v_scr[pl.ds(r0, _ROWS_PER_TILE)]
    if2 = lax.dot_general(m_full, t_tile, (((1,), (2,)), ((), ())),
                          preferred_element_type=jnp.float32
                          ).astype(jnp.bfloat16)
    img_f = if2.reshape(_TILE_TOKENS, _C)

    feat = jnp.concatenate([img_f, coords_ref[0]], axis=1)

    def ldot(x, w_ref, b_ref):
        r = lax.dot_general(x, w_ref[...], (((1,), (0,)), ((), ())),
                            preferred_element_type=jnp.float32) + b_ref[...]
        return r

    def ldot16(x, w_ref, b_ref):
        return ldot(x, w_ref, b_ref).astype(jnp.bfloat16)

    h = jax.nn.relu(ldot16(feat, w0a_ref, b0a_ref))
    h = jax.nn.relu(ldot16(h, w0b_ref, b0b_ref))
    s0 = ldot(h, wh0_ref, bh0_ref)

    s0_t = jnp.transpose(s0.reshape(_W, _ROWS_PER_TILE), (1, 0))
    coarse_ref[...] = s0_t[None]

    @pl.when(i < _FINE_TILES)
    def _fine():
        s_rest = ldot16(h, whr_ref, bhr_ref)
        fine_in = jnp.concatenate([feat, s_rest], axis=1)
        y = jax.nn.relu(ldot16(fine_in, w1a_ref, b1a_ref))
        y = jax.nn.relu(ldot16(y, w1b_ref, b1b_ref))
        z = jax.nn.relu(ldot16(y, w3a_ref, b3a_ref))
        z = ldot(z, w3b_ref, b3b_ref)
        z_t = jnp.transpose(z.reshape(_W, _ROWS_PER_TILE), (1, 0))
        tid = i * _TILE_TOKENS \
            + lax.broadcasted_iota(jnp.int32, (_ROWS_PER_TILE, _W), 0) * _W \
            + lax.broadcasted_iota(jnp.int32, (_ROWS_PER_TILE, _W), 1)
        fine_ref[...] = jnp.where(tid < _K, z_t, s0_t)[None]

    @pl.when(i >= _FINE_TILES)
    def _copy():
        fine_ref[...] = s0_t[None]


def _fold(lin, bn):
    scale = bn['g'] / jnp.sqrt(bn['v'] + _EPS)
    w = lin['W'] * scale[None, :]
    b = (lin['b'] - bn['m']) * scale + bn['be']
    return w.astype(jnp.bfloat16), b.astype(jnp.float32).reshape(1, -1)


def _coords_table():
    gy = jnp.linspace(-1.0, 1.0, _H)
    gx = jnp.linspace(-1.0, 1.0, _W)
    yy = jnp.broadcast_to(gy[:, None], (_H, _W))
    xx = jnp.broadcast_to(gx[None, :], (_H, _W))
    coords = jnp.stack([yy, xx], axis=-1)
    freqs = 2.0 ** jnp.linspace(0.0, _MAX_FREQ, _NUM_FREQ)
    si = (2.0 * math.pi * coords[..., None] * freqs).reshape(_H, _W, -1)
    enc = jnp.concatenate([jnp.sin(si), jnp.sin(si + math.pi / 2.0), coords],
                          axis=-1).astype(jnp.bfloat16)
    t = enc.reshape(_TILES_PER_BATCH, _ROWS_PER_TILE, _W, _POS_DIM)
    return jnp.transpose(t, (0, 2, 1, 3)).reshape(
        _TILES_PER_BATCH, _TILE_TOKENS, _POS_DIM)


@jax.jit
def _run(image_embedding, image_pe, params):
    p = params
    w0a, b0a = _fold(p['l0a'], p['bn0a'])
    w0b, b0b = _fold(p['l0b'], p['bn0b'])
    wh = p['head']['W']
    bh = p['head']['b'].astype(jnp.float32)
    wh0, bh0 = wh[:, 0:1].astype(jnp.bfloat16), bh[0:1].reshape(1, 1)
    whr, bhr = wh[:, 1:].astype(jnp.bfloat16), bh[1:].reshape(1, -1)
    w1a, b1a = _fold(p['l1a'], p['bn1a'])
    w1b, b1b = _fold(p['l1b'], p['bn1b'])
    w3a, b3a = _fold(p['l3a'], p['bn3a'])
    w3b = p['l3b']['W'].astype(jnp.bfloat16)
    b3b = p['l3b']['b'].astype(jnp.float32).reshape(1, 1)

    m = jax.image.resize(jnp.eye(_H0, dtype=jnp.float32), (_H, _H0),
                         method='bilinear').astype(jnp.bfloat16)
    coords = _coords_table()
    emb = image_embedding.astype(jnp.float32)
    pe = image_pe.astype(jnp.float32)

    def whole(a):
        return pl.BlockSpec(a.shape, lambda i: (0,) * a.ndim)

    grid = (_NUM_TILES,)
    in_specs = [
        pl.BlockSpec((1, 32, _H0, _H0), lambda i: (i // _TILES_PER_BATCH, 0, 0, 0)),
        pl.BlockSpec((1, 128, _H0, _H0), lambda i: (i // _TILES_PER_BATCH, 0, 0, 0)),
        pl.BlockSpec((1, _TILE_TOKENS, _POS_DIM),
                     lambda i: (lax.rem(i, _TILES_PER_BATCH), 0, 0)),
        whole(m),
        whole(w0a), whole(b0a), whole(w0b), whole(b0b),
        whole(wh0), whole(bh0), whole(whr), whole(bhr),
        whole(w1a), whole(b1a), whole(w1b), whole(b1b),
        whole(w3a), whole(b3a), whole(w3b), whole(b3b),
    ]
    out_spec = pl.BlockSpec((1, _ROWS_PER_TILE, _W),
                            lambda i: (i // _TILES_PER_BATCH,
                                       lax.rem(i, _TILES_PER_BATCH), 0))
    coarse, fine = pl.pallas_call(
        _tile_kernel,
        grid=grid,
        in_specs=in_specs,
        out_specs=[out_spec, out_spec],
        out_shape=[jax.ShapeDtypeStruct((_B, _H, _W), jnp.float32)] * 2,
        scratch_shapes=[pltpu.VMEM((_H, _C, _H0), jnp.bfloat16)],
    )(emb, pe, coords, m, w0a, b0a, w0b, b0b, wh0, bh0, whr, bhr,
      w1a, b1a, w1b, b1b, w3a, b3a, w3b, b3b)
    return (coarse.reshape(_B, 1, _H, _W), fine.reshape(_B, 1, _H, _W))


def kernel(image_embedding, image_pe, params, original_shape):
    del original_shape
    return _run(image_embedding, image_pe, params)
